# Initial kernel scaffold; baseline (speedup 1.0000x reference)
import jax
import jax.numpy as jnp
from jax.experimental import pallas as pl


def kernel(X, tables, W_lin, b_lin, W1, b1, W2, b2, W_out):
    raise NotImplementedError("write your pallas kernel here")


# speedup vs baseline: 2.1211x; 2.1211x over previous
import jax
import jax.numpy as jnp
from jax import lax
from jax.experimental import pallas as pl
from jax.experimental.pallas import tpu as pltpu
from jax.experimental.pallas import tpu_sc as plsc

_B = 16384
_N_SPARSE = 26
_N_DENSE = 13
_VOCAB = 100000
_EDIM = 16
_HID1 = 256
_HID2 = 128
_N_FEAT = _N_SPARSE + _N_DENSE
_EFLAT = _N_SPARSE * _EDIM

_NC, _NS = 2, 16
_NW = _NC * _NS
_ROWS = _B * _N_SPARSE
_RPW = _ROWS // _NW
_STREAM = 128
_SPC = 8
_CHUNK = _STREAM * _SPC
_NCHUNK = _RPW // _CHUNK
_NSTREAMS = _RPW // _STREAM


def _gather_body(tbl, idx, out, idx_v, rows_v, sem):
    wid = lax.axis_index("s") * _NC + lax.axis_index("c")
    base = wid * _RPW
    pltpu.sync_copy(idx.at[wid], idx_v)

    def chunk(c, carry):
        cps = [
            pltpu.async_copy(
                tbl.at[idx_v.at[c * _SPC + j]],
                rows_v.at[pl.ds(j * _STREAM, _STREAM)],
                sem,
            )
            for j in range(_SPC)
        ]
        for cp in cps:
            cp.wait()
        pltpu.sync_copy(rows_v, out.at[pl.ds(base + c * _CHUNK, _CHUNK)])
        return carry

    lax.fori_loop(0, _NCHUNK, chunk, 0)


def _sc_gather(tbl_flat, idx):
    mesh = plsc.VectorSubcoreMesh(core_axis_name="c", subcore_axis_name="s")
    f = pl.kernel(
        _gather_body,
        mesh=mesh,
        out_type=jax.ShapeDtypeStruct((_ROWS, _EDIM), jnp.float32),
        scratch_types=[
            pltpu.VMEM((_NSTREAMS, _STREAM), jnp.int32),
            pltpu.VMEM((_CHUNK, _EDIM), jnp.float32),
            pltpu.SemaphoreType.DMA,
        ],
        compiler_params=pltpu.CompilerParams(use_tc_tiling_on_sc=False),
    )
    return f(tbl_flat, idx)


_BS = 1024


def _mlp_body(emb, x, w1e, w1d, b1, w2, b2, wout, wlin, blin, out):
    xv = x[...]
    h1 = jnp.dot(emb[...], w1e[...], preferred_element_type=jnp.float32)
    h1 = h1 + jnp.dot(xv[:, _N_SPARSE:], w1d[...],
                      preferred_element_type=jnp.float32)
    h1 = jnp.maximum(h1 + b1[...], 0.0)
    h2 = jnp.dot(h1, w2[...], preferred_element_type=jnp.float32)
    h2 = jnp.maximum(h2 + b2[...], 0.0)
    dlogit = jnp.sum(h2 * wout[...], axis=1, keepdims=True)
    wlogit = jnp.sum(xv * wlin[...], axis=1, keepdims=True) + blin[...]
    wlogit = jnp.maximum(wlogit, 0.0)
    z = wlogit + dlogit
    out[...] = 1.0 / (1.0 + jnp.exp(-z))


def _mlp(emb, X, w1e, w1d, b1, w2, b2, wout, wlin, blin):
    grid = (_B // _BS,)
    full = lambda r, c: pl.BlockSpec((r, c), lambda i: (0, 0))
    return pl.pallas_call(
        _mlp_body,
        grid=grid,
        in_specs=[
            pl.BlockSpec((_BS, _EFLAT), lambda i: (i, 0)),
            pl.BlockSpec((_BS, _N_FEAT), lambda i: (i, 0)),
            full(_EFLAT, _HID1),
            full(_N_DENSE, _HID1),
            full(1, _HID1),
            full(_HID1, _HID2),
            full(1, _HID2),
            full(1, _HID2),
            full(1, _N_FEAT),
            full(1, 1),
        ],
        out_specs=pl.BlockSpec((_BS, 1), lambda i: (i, 0)),
        out_shape=jax.ShapeDtypeStruct((_B, 1), jnp.float32),
        compiler_params=pltpu.CompilerParams(
            dimension_semantics=("arbitrary",)),
    )(emb, X, w1e, w1d, b1, w2, b2, wout, wlin, blin)


def kernel(X, tables, W_lin, b_lin, W1, b1, W2, b2, W_out):
    offs = (jnp.arange(_N_SPARSE, dtype=jnp.int32) * _VOCAB)[None, :]
    idx = X[:, :_N_SPARSE].astype(jnp.int32) + offs
    idx = idx.reshape(_NW, _NSTREAMS, _STREAM)
    tbl_flat = tables.reshape(_N_SPARSE * _VOCAB, _EDIM)
    emb = _sc_gather(tbl_flat, idx).reshape(_B, _EFLAT)
    out = _mlp(
        emb, X,
        W1[:, :_EFLAT].T, W1[:, _EFLAT:].T, b1[None, :],
        W2.T, b2[None, :], W_out, W_lin, b_lin[None, :],
    )
    return out
